# Initial kernel scaffold; baseline (speedup 1.0000x reference)
import jax
import jax.numpy as jnp
from jax.experimental import pallas as pl


def kernel(x, edge_index, W1, W2):
    raise NotImplementedError("write your pallas kernel here")


# speedup vs baseline: 7.2592x; 7.2592x over previous
import functools

import jax
import jax.numpy as jnp
from jax import lax
from jax.experimental import pallas as pl
from jax.experimental.pallas import tpu as pltpu
from jax.experimental.pallas import tpu_sc as plsc

NC = 2
NS = 16
C = 80


def _segment_sum_sc(h, src3d, dst3d, zeros):
  n, d = h.shape
  nw, kpt, c = src3d.shape
  npad = ((n + 8 * NS - 1) // (8 * NS)) * (8 * NS)
  rpt = npad // NS
  mesh = plsc.VectorSubcoreMesh(core_axis_name="c", subcore_axis_name="s")

  @functools.partial(
      pl.kernel,
      out_type=jax.ShapeDtypeStruct((NC * npad, d), jnp.float32),
      mesh=mesh,
      scratch_types=[
          pltpu.VMEM_SHARED((npad, d), jnp.float32),
          pltpu.VMEM((kpt, c), jnp.int32),
          pltpu.VMEM((kpt, c), jnp.int32),
          pltpu.VMEM((c, d), jnp.float32),
          pltpu.SemaphoreType.DMA,
      ],
  )
  def seg_sum(h_hbm, src_hbm, dst_hbm, zeros_hbm, out_hbm, acc, src_idx,
              dst_idx, rows, sem):
    cid = lax.axis_index("c")
    sid = lax.axis_index("s")
    wid = cid * NS + sid
    rz = sid * rpt
    pltpu.sync_copy(zeros_hbm.at[pl.ds(rz, rpt)], acc.at[pl.ds(rz, rpt)])
    pltpu.sync_copy(src_hbm.at[wid], src_idx)
    pltpu.sync_copy(dst_hbm.at[wid], dst_idx)
    plsc.subcore_barrier()

    def chunk(i, carry):
      pltpu.async_copy(h_hbm.at[src_idx.at[i]], rows, sem).wait()
      pltpu.sync_copy(rows, acc.at[dst_idx.at[i]], add=True)
      return carry

    lax.fori_loop(0, kpt, chunk, 0)
    plsc.subcore_barrier()
    pltpu.sync_copy(acc.at[pl.ds(rz, rpt)],
                    out_hbm.at[pl.ds(cid * npad + rz, rpt)])

  return seg_sum(h, src3d, dst3d, zeros), npad


def _mm_kernel(x_ref, w_ref, o_ref):
  o_ref[...] = jnp.dot(x_ref[...], w_ref[...],
                       preferred_element_type=jnp.float32,
                       precision=lax.Precision.HIGHEST)


def _relu_mm_kernel(n, p_ref, w_ref, o_ref):
  h = jnp.maximum(p_ref[0, :n] + p_ref[1, :n], 0.0)
  o_ref[...] = jnp.dot(h, w_ref[...],
                       preferred_element_type=jnp.float32,
                       precision=lax.Precision.HIGHEST)


def _add_log_softmax_kernel(n, p_ref, o_ref):
  h = p_ref[0, :n] + p_ref[1, :n]
  m = jnp.max(h, axis=1, keepdims=True)
  lse = jnp.log(jnp.sum(jnp.exp(h - m), axis=1, keepdims=True)) + m
  o_ref[...] = h - lse


def kernel(x, edge_index, W1, W2):
  n, d = x.shape
  e = edge_index.shape[1]
  nw = NC * NS
  kpt = e // (C * nw)
  dst3d = edge_index[0].reshape(nw, kpt, C)
  src3d = edge_index[1].reshape(nw, kpt, C)
  npad = ((n + 8 * NS - 1) // (8 * NS)) * (8 * NS)
  zeros = jnp.zeros((npad, d), jnp.float32)

  h1 = pl.pallas_call(
      _mm_kernel,
      out_shape=jax.ShapeDtypeStruct((n, d), jnp.float32),
  )(x, W1)
  p1, _ = _segment_sum_sc(h1, src3d, dst3d, zeros)
  h2 = pl.pallas_call(
      functools.partial(_relu_mm_kernel, n),
      out_shape=jax.ShapeDtypeStruct((n, d), jnp.float32),
  )(p1.reshape(NC, npad, d), W2)
  p2, _ = _segment_sum_sc(h2, src3d, dst3d, zeros)
  return pl.pallas_call(
      functools.partial(_add_log_softmax_kernel, n),
      out_shape=jax.ShapeDtypeStruct((n, d), jnp.float32),
  )(p2.reshape(NC, npad, d))


# speedup vs baseline: 11.7115x; 1.6133x over previous
import functools

import jax
import jax.numpy as jnp
from jax import lax
from jax.experimental import pallas as pl
from jax.experimental.pallas import tpu as pltpu
from jax.experimental.pallas import tpu_sc as plsc

NC = 2
NS = 16
C = 125
G = 16


def _segment_sum_sc(h, src3d, dst3d, zeros):
  n, d = h.shape
  nw, kpt, c = src3d.shape
  npad = ((n + 8 * NS - 1) // (8 * NS)) * (8 * NS)
  rpt = npad // NS
  mesh = plsc.VectorSubcoreMesh(core_axis_name="c", subcore_axis_name="s")

  @functools.partial(
      pl.kernel,
      out_type=jax.ShapeDtypeStruct((NC * npad, d), jnp.float32),
      mesh=mesh,
      scratch_types=[
          pltpu.VMEM_SHARED((npad, d), jnp.float32),
          pltpu.VMEM((G, c), jnp.int32),
          pltpu.VMEM((G, c), jnp.int32),
          pltpu.VMEM((G, c), jnp.int32),
          pltpu.VMEM((G, c), jnp.int32),
          pltpu.VMEM((c, d), jnp.float32),
          pltpu.VMEM((c, d), jnp.float32),
          pltpu.SemaphoreType.DMA,
          pltpu.SemaphoreType.DMA,
          pltpu.SemaphoreType.DMA,
          pltpu.SemaphoreType.DMA,
      ],
  )
  def seg_sum(h_hbm, src_hbm, dst_hbm, zeros_hbm, out_hbm, acc,
              srcg0, dstg0, srcg1, dstg1, rows0, rows1,
              sem0, sem1, semi0, semi1):
    cid = lax.axis_index("c")
    sid = lax.axis_index("s")
    wid = cid * NS + sid
    ngroups = kpt // G
    npairs = G // 2
    pltpu.async_copy(src_hbm.at[wid, pl.ds(0, G)], srcg0, semi0)
    pltpu.async_copy(dst_hbm.at[wid, pl.ds(0, G)], dstg0, semi0)
    rz = sid * rpt
    pltpu.sync_copy(zeros_hbm.at[pl.ds(rz, rpt)], acc.at[pl.ds(rz, rpt)])
    plsc.subcore_barrier()

    for g in range(ngroups):
      srcg, dstg, semi = ((srcg0, dstg0, semi0) if g % 2 == 0 else
                          (srcg1, dstg1, semi1))
      pltpu.make_async_copy(src_hbm.at[wid, pl.ds(g * G, G)], srcg,
                            semi).wait()
      pltpu.make_async_copy(dst_hbm.at[wid, pl.ds(g * G, G)], dstg,
                            semi).wait()
      if g + 1 < ngroups:
        nsrc, ndst, nsem = ((srcg1, dstg1, semi1) if g % 2 == 0 else
                            (srcg0, dstg0, semi0))
        pltpu.async_copy(src_hbm.at[wid, pl.ds((g + 1) * G, G)], nsrc, nsem)
        pltpu.async_copy(dst_hbm.at[wid, pl.ds((g + 1) * G, G)], ndst, nsem)

      pltpu.async_copy(h_hbm.at[srcg.at[0]], rows0, sem0)

      def pair(j, carry, srcg=srcg, dstg=dstg):
        k = j * 2
        pltpu.make_async_copy(h_hbm.at[srcg.at[k]], rows0, sem0).wait()
        d1 = pltpu.async_copy(h_hbm.at[srcg.at[k + 1]], rows1, sem1)
        pltpu.sync_copy(rows0, acc.at[dstg.at[k]], add=True)

        @pl.when(j + 1 < npairs)
        def _():
          pltpu.async_copy(h_hbm.at[srcg.at[k + 2]], rows0, sem0)

        d1.wait()
        pltpu.sync_copy(rows1, acc.at[dstg.at[k + 1]], add=True)
        return carry

      lax.fori_loop(0, npairs, pair, 0)
    plsc.subcore_barrier()
    pltpu.sync_copy(acc.at[pl.ds(rz, rpt)],
                    out_hbm.at[pl.ds(cid * npad + rz, rpt)])

  return seg_sum(h, src3d, dst3d, zeros), npad


def _mm_kernel(x_ref, w_ref, o_ref):
  o_ref[...] = jnp.dot(x_ref[...], w_ref[...],
                       preferred_element_type=jnp.float32,
                       precision=lax.Precision.HIGHEST)


def _relu_mm_kernel(n, p_ref, w_ref, o_ref):
  h = jnp.maximum(p_ref[0, :n] + p_ref[1, :n], 0.0)
  o_ref[...] = jnp.dot(h, w_ref[...],
                       preferred_element_type=jnp.float32,
                       precision=lax.Precision.HIGHEST)


def _add_log_softmax_kernel(n, p_ref, o_ref):
  h = p_ref[0, :n] + p_ref[1, :n]
  m = jnp.max(h, axis=1, keepdims=True)
  lse = jnp.log(jnp.sum(jnp.exp(h - m), axis=1, keepdims=True)) + m
  o_ref[...] = h - lse


def kernel(x, edge_index, W1, W2):
  n, d = x.shape
  e = edge_index.shape[1]
  nw = NC * NS
  kpt = e // (C * nw)
  dst3d = edge_index[0].reshape(nw, kpt, C)
  src3d = edge_index[1].reshape(nw, kpt, C)
  npad = ((n + 8 * NS - 1) // (8 * NS)) * (8 * NS)
  zeros = jnp.zeros((npad, d), jnp.float32)

  h1 = pl.pallas_call(
      _mm_kernel,
      out_shape=jax.ShapeDtypeStruct((n, d), jnp.float32),
  )(x, W1)
  p1, _ = _segment_sum_sc(h1, src3d, dst3d, zeros)
  h2 = pl.pallas_call(
      functools.partial(_relu_mm_kernel, n),
      out_shape=jax.ShapeDtypeStruct((n, d), jnp.float32),
  )(p1.reshape(NC, npad, d), W2)
  p2, _ = _segment_sum_sc(h2, src3d, dst3d, zeros)
  return pl.pallas_call(
      functools.partial(_add_log_softmax_kernel, n),
      out_shape=jax.ShapeDtypeStruct((n, d), jnp.float32),
  )(p2.reshape(NC, npad, d))
